# baseline (speedup 1.0000x reference)
import jax
import jax.numpy as jnp
from jax.experimental import pallas as pl
from jax.experimental.pallas import tpu as pltpu

_ROWS = 8192
_COLS = 128
_BLOCK_ROWS = 4096


def _copy_block(t_hbm, o_ref, sem):
    i = pl.program_id(0)
    cp = pltpu.make_async_copy(
        t_hbm.at[pl.ds(i * _BLOCK_ROWS, _BLOCK_ROWS), :], o_ref, sem
    )
    cp.start()
    cp.wait()


def kernel(x, pos_table):
    del x
    n_blocks = _ROWS // _BLOCK_ROWS
    return pl.pallas_call(
        _copy_block,
        out_shape=jax.ShapeDtypeStruct((_ROWS, _COLS), pos_table.dtype),
        grid=(n_blocks,),
        in_specs=[pl.BlockSpec(memory_space=pl.ANY)],
        out_specs=pl.BlockSpec((_BLOCK_ROWS, _COLS), lambda i: (i, 0)),
        scratch_shapes=[pltpu.SemaphoreType.DMA],
    )(pos_table)
